# baseline (speedup 1.0000x reference)
import functools

import jax
import jax.numpy as jnp
from jax import lax
from jax.experimental import pallas as pl
from jax.experimental.pallas import tpu as pltpu
from jax.experimental.pallas import tpu_sc as plsc

_VOCAB = 100000
_DM = 1024
_N_BYTES = 16
_BYTE_VOCAB = 256

_NW = 32
_CHUNK = 32


def _sc_token_gather(ids_flat, token_table):
    n = ids_flat.shape[0]
    bpw = n // _NW
    nch = bpw // _CHUNK
    mesh = plsc.VectorSubcoreMesh(core_axis_name="c", subcore_axis_name="s")

    @functools.partial(
        pl.kernel,
        out_type=jax.ShapeDtypeStruct((n, _DM), jnp.float32),
        mesh=mesh,
        scratch_types=[
            pltpu.VMEM((bpw,), jnp.int32),
            pltpu.VMEM((_CHUNK, _DM), jnp.float32),
            pltpu.VMEM((_CHUNK, _DM), jnp.float32),
            pltpu.VMEM((_CHUNK, _DM), jnp.float32),
            pltpu.SemaphoreType.DMA,
            pltpu.SemaphoreType.DMA,
            pltpu.SemaphoreType.DMA,
            pltpu.SemaphoreType.DMA,
            pltpu.SemaphoreType.DMA,
            pltpu.SemaphoreType.DMA,
        ],
    )
    def k(ids_hbm, table_hbm, out_hbm, idx_v,
          rows0, rows1, rows2, gs0, gs1, gs2, ws0, ws1, ws2):
        cid = lax.axis_index("c")
        sid = lax.axis_index("s")
        wid = sid * 2 + cid
        base = wid * bpw
        pltpu.sync_copy(ids_hbm.at[pl.ds(base, bpw)], idx_v)
        rows = (rows0, rows1, rows2)
        gs = (gs0, gs1, gs2)
        ws = (ws0, ws1, ws2)

        def start_g(i):
            return pltpu.async_copy(
                table_hbm.at[idx_v.at[pl.ds(i * _CHUNK, _CHUNK)]],
                rows[i % 3],
                gs[i % 3],
            )

        gh = {0: start_g(0)}
        wh = {}
        for i in range(nch):
            if i + 1 < nch:
                if i - 2 >= 0:
                    wh[i - 2].wait()
                gh[i + 1] = start_g(i + 1)
            gh[i].wait()
            off = pl.multiple_of(base + i * _CHUNK, _CHUNK)
            wh[i] = pltpu.async_copy(
                rows[i % 3], out_hbm.at[pl.ds(off, _CHUNK)], ws[i % 3]
            )
        for i in range(max(0, nch - 2), nch):
            wh[i].wait()

    return k(ids_flat, token_table)


_TOK_BLK = 1024


def _tc_combine_body(bytes_t_ref, gathered_ref, pos_ref, btab_ref, out_ref):
    viota = lax.broadcasted_iota(jnp.int32, (_BYTE_VOCAB, _TOK_BLK), 0)
    cnt_t = jnp.zeros((_BYTE_VOCAB, _TOK_BLK), jnp.float32)
    for j in range(_N_BYTES):
        b = bytes_t_ref[j : j + 1, :]
        cnt_t = cnt_t + (b == viota).astype(jnp.float32)
    bag = lax.dot_general(
        cnt_t, btab_ref[...], (((0,), (0,)), ((), ())),
        preferred_element_type=jnp.float32,
    )
    out_ref[...] = gathered_ref[...] + pos_ref[...].astype(jnp.float32) + bag


def _tc_combine_into(big, bytes_t, gathered, pos_table, byte_table, b0, out_shape):
    n = gathered.shape[0]
    s = pos_table.shape[0]
    pos_blocks = s // _TOK_BLK
    nb = n // s
    tok_l = lambda p, b: b * pos_blocks + p
    tok_g = lambda p, b: (b0 + b) * pos_blocks + p
    in_specs = [
        pl.BlockSpec((_N_BYTES, _TOK_BLK), lambda p, b: (0, tok_l(p, b))),
        pl.BlockSpec((_TOK_BLK, _DM), lambda p, b: (tok_l(p, b), 0)),
        pl.BlockSpec((_TOK_BLK, _DM), lambda p, b: (p, 0)),
        pl.BlockSpec((_BYTE_VOCAB, _DM), lambda p, b: (0, 0)),
    ]
    args = (bytes_t, gathered, pos_table, byte_table)
    if big is None:
        body = _tc_combine_body
        aliases = {}
    else:
        body = lambda big_ref, bt, g, pos, btab, out: _tc_combine_body(
            bt, g, pos, btab, out
        )
        in_specs = [pl.BlockSpec(memory_space=pl.ANY)] + in_specs
        args = (big,) + args
        aliases = {0: 0}
    return pl.pallas_call(
        body,
        grid=(pos_blocks, nb),
        in_specs=in_specs,
        out_specs=pl.BlockSpec((_TOK_BLK, _DM), lambda p, b: (tok_g(p, b), 0)),
        out_shape=jax.ShapeDtypeStruct(out_shape, jnp.float32),
        input_output_aliases=aliases,
    )(*args)


_NSPLIT = 4


def kernel(input_ids, input_bytes, token_table, pos_table, byte_table):
    b, s = input_ids.shape
    n = b * s
    nb_h = b // _NSPLIT
    n_h = nb_h * s
    gathered = [
        _sc_token_gather(
            input_ids[h * nb_h : (h + 1) * nb_h].reshape(n_h), token_table
        )
        for h in range(_NSPLIT)
    ]
    pos_table = pos_table.astype(jnp.bfloat16)
    big = None
    for h in range(_NSPLIT):
        bytes_t = (
            input_bytes[h * nb_h : (h + 1) * nb_h].reshape(n_h, _N_BYTES).T
        )
        big = _tc_combine_into(
            big, bytes_t, gathered[h], pos_table, byte_table, h * nb_h, (n, _DM)
        )
    return big.reshape(b, s, _DM)


# speedup vs baseline: 1.0834x; 1.0834x over previous
import functools

import jax
import jax.numpy as jnp
from jax import lax
from jax.experimental import pallas as pl
from jax.experimental.pallas import tpu as pltpu
from jax.experimental.pallas import tpu_sc as plsc

_VOCAB = 100000
_DM = 1024
_N_BYTES = 16
_BYTE_VOCAB = 256

_NW = 32
_CHUNK = 32


def _sc_token_gather(ids_flat, token_table):
    n = ids_flat.shape[0]
    bpw = n // _NW
    nch = bpw // _CHUNK
    mesh = plsc.VectorSubcoreMesh(core_axis_name="c", subcore_axis_name="s")

    @functools.partial(
        pl.kernel,
        out_type=jax.ShapeDtypeStruct((n, _DM), jnp.float32),
        mesh=mesh,
        scratch_types=[
            pltpu.VMEM((bpw,), jnp.int32),
            pltpu.VMEM((_CHUNK, _DM), jnp.float32),
            pltpu.VMEM((_CHUNK, _DM), jnp.float32),
            pltpu.VMEM((_CHUNK, _DM), jnp.float32),
            pltpu.SemaphoreType.DMA,
            pltpu.SemaphoreType.DMA,
            pltpu.SemaphoreType.DMA,
            pltpu.SemaphoreType.DMA,
            pltpu.SemaphoreType.DMA,
            pltpu.SemaphoreType.DMA,
        ],
    )
    def k(ids_hbm, table_hbm, out_hbm, idx_v,
          rows0, rows1, rows2, gs0, gs1, gs2, ws0, ws1, ws2):
        cid = lax.axis_index("c")
        sid = lax.axis_index("s")
        wid = sid * 2 + cid
        base = wid * bpw
        pltpu.sync_copy(ids_hbm.at[pl.ds(base, bpw)], idx_v)
        rows = (rows0, rows1, rows2)
        gs = (gs0, gs1, gs2)
        ws = (ws0, ws1, ws2)

        def start_g(i):
            return pltpu.async_copy(
                table_hbm.at[idx_v.at[pl.ds(i * _CHUNK, _CHUNK)]],
                rows[i % 3],
                gs[i % 3],
            )

        gh = {0: start_g(0)}
        if nch > 1:
            gh[1] = start_g(1)
        wh = {}
        for i in range(nch):
            if i + 2 < nch:
                if i - 1 >= 0:
                    wh[i - 1].wait()
                gh[i + 2] = start_g(i + 2)
            gh[i].wait()
            off = pl.multiple_of(base + i * _CHUNK, _CHUNK)
            wh[i] = pltpu.async_copy(
                rows[i % 3], out_hbm.at[pl.ds(off, _CHUNK)], ws[i % 3]
            )
        for i in range(max(0, nch - 3), nch):
            wh[i].wait()

    return k(ids_flat, token_table)


_TOK_BLK = 2048


def _tc_combine_body(bytes_t_ref, gathered_ref, pos_ref, btab_ref, out_ref):
    viota = lax.broadcasted_iota(jnp.int32, (_BYTE_VOCAB, _TOK_BLK), 0)
    cnt_t = jnp.zeros((_BYTE_VOCAB, _TOK_BLK), jnp.float32)
    for j in range(_N_BYTES):
        b = bytes_t_ref[j : j + 1, :]
        cnt_t = cnt_t + (b == viota).astype(jnp.float32)
    bag = lax.dot_general(
        cnt_t, btab_ref[...], (((0,), (0,)), ((), ())),
        preferred_element_type=jnp.float32,
    )
    out_ref[...] = gathered_ref[...] + pos_ref[...].astype(jnp.float32) + bag


def _tc_combine_into(big, bytes_t, gathered, pos_table, byte_table, b0, out_shape):
    n = gathered.shape[0]
    s = pos_table.shape[0]
    pos_blocks = s // _TOK_BLK
    nb = n // s
    tok_l = lambda p, b: b * pos_blocks + p
    tok_g = lambda p, b: (b0 + b) * pos_blocks + p
    in_specs = [
        pl.BlockSpec((_N_BYTES, _TOK_BLK), lambda p, b: (0, tok_l(p, b))),
        pl.BlockSpec((_TOK_BLK, _DM), lambda p, b: (tok_l(p, b), 0)),
        pl.BlockSpec((_TOK_BLK, _DM), lambda p, b: (p, 0)),
        pl.BlockSpec((_BYTE_VOCAB, _DM), lambda p, b: (0, 0)),
    ]
    args = (bytes_t, gathered, pos_table, byte_table)
    if big is None:
        body = _tc_combine_body
        aliases = {}
    else:
        body = lambda big_ref, bt, g, pos, btab, out: _tc_combine_body(
            bt, g, pos, btab, out
        )
        in_specs = [pl.BlockSpec(memory_space=pl.ANY)] + in_specs
        args = (big,) + args
        aliases = {0: 0}
    return pl.pallas_call(
        body,
        grid=(pos_blocks, nb),
        in_specs=in_specs,
        out_specs=pl.BlockSpec((_TOK_BLK, _DM), lambda p, b: (tok_g(p, b), 0)),
        out_shape=jax.ShapeDtypeStruct(out_shape, jnp.float32),
        input_output_aliases=aliases,
    )(*args)


_NSPLIT = 2


def kernel(input_ids, input_bytes, token_table, pos_table, byte_table):
    b, s = input_ids.shape
    n = b * s
    nb_h = b // _NSPLIT
    n_h = nb_h * s
    gathered = [
        _sc_token_gather(
            input_ids[h * nb_h : (h + 1) * nb_h].reshape(n_h), token_table
        )
        for h in range(_NSPLIT)
    ]
    pos_table = pos_table.astype(jnp.bfloat16)
    big = None
    for h in range(_NSPLIT):
        bytes_t = (
            input_bytes[h * nb_h : (h + 1) * nb_h].reshape(n_h, _N_BYTES).T
        )
        big = _tc_combine_into(
            big, bytes_t, gathered[h], pos_table, byte_table, h * nb_h, (n, _DM)
        )
    return big.reshape(b, s, _DM)
